# Initial kernel scaffold; baseline (speedup 1.0000x reference)
import jax
import jax.numpy as jnp
from jax.experimental import pallas as pl


def kernel(drug_x, drug_edge_index, drug_batch_ids, conf_x, conf_edge_index, conf_batch_ids, Wd0, bd0, Wd1, bd1, Wd2, bd2, Wp0, bp0, Wp1, bp1, Wp2, bp2, Wq, bq, Wk, bk, Wv, bv, Wh1, bh1, Wh2, bh2):
    raise NotImplementedError("write your pallas kernel here")


# speedup vs baseline: 3.0729x; 3.0729x over previous
import jax
import jax.numpy as jnp
from jax.experimental import pallas as pl

ND = 50000; ED = 800000; NP_ = 10000; EP = 160000; NC = 4; B = 256; DD = 78; DP = 128; H = 128


def _gcn_conv(x, edge_index, W, b, n):
    x = x @ W
    src = edge_index[0]
    dst = edge_index[1]
    deg = jnp.zeros((n,), x.dtype).at[dst].add(1.0) + 1.0
    dis = 1.0 / jnp.sqrt(deg)
    z = x * dis[:, None]
    s = jnp.zeros_like(x).at[dst].add(z[src])
    out = dis[:, None] * s + (dis * dis)[:, None] * x
    return out + b


def _pool(x, seg, num_graphs):
    s = jax.ops.segment_sum(x, seg, num_segments=num_graphs)
    c = jax.ops.segment_sum(jnp.ones((x.shape[0], 1), x.dtype), seg, num_segments=num_graphs)
    return s / jnp.maximum(c, 1.0)


def _tail_kernel(drug_ref, conf_ref, Wq_ref, bq_ref, Wk_ref, bk_ref, Wv_ref, bv_ref,
                 Wh1_ref, bh1_ref, Wh2_ref, bh2_ref, logits_ref, attn_ref):
    drug = drug_ref[...]
    q = drug @ Wq_ref[...] + bq_ref[...]
    scores = []
    vals = []
    for c in range(NC):
        kc = conf_ref[c] @ Wk_ref[...] + bk_ref[...]
        vc = conf_ref[c] @ Wv_ref[...] + bv_ref[...]
        scores.append(jnp.sum(q * kc, axis=1) / (H ** 0.5))
        vals.append(vc)
    sc = jnp.stack(scores, axis=1)
    m = jnp.max(sc, axis=1, keepdims=True)
    e = jnp.exp(sc - m)
    attn = e / jnp.sum(e, axis=1, keepdims=True)
    attended = sum(vals[c] * attn[:, c:c + 1] for c in range(NC))
    h1 = jnp.maximum(drug @ Wh1_ref[:H] + attended @ Wh1_ref[H:] + bh1_ref[...], 0.0)
    logits_ref[...] = h1 @ Wh2_ref[...] + bh2_ref[...]
    attn_ref[...] = attn


def kernel(drug_x, drug_edge_index, drug_batch_ids, conf_x, conf_edge_index, conf_batch_ids,
           Wd0, bd0, Wd1, bd1, Wd2, bd2, Wp0, bp0, Wp1, bp1, Wp2, bp2,
           Wq, bq, Wk, bk, Wv, bv, Wh1, bh1, Wh2, bh2):
    h = drug_x
    for (W, b) in ((Wd0, bd0), (Wd1, bd1), (Wd2, bd2)):
        h = jax.nn.relu(_gcn_conv(h, drug_edge_index, W, b, ND))
    drug_emb = _pool(h, drug_batch_ids, B)
    conf_embs = []
    for i in range(NC):
        hp = conf_x[i]
        for (W, b) in ((Wp0, bp0), (Wp1, bp1), (Wp2, bp2)):
            hp = jax.nn.relu(_gcn_conv(hp, conf_edge_index[i], W, b, NP_))
        conf_embs.append(_pool(hp, conf_batch_ids, B))
    conf_embeddings = jnp.stack(conf_embs, axis=0)

    logits2, attn = pl.pallas_call(
        _tail_kernel,
        out_shape=(
            jax.ShapeDtypeStruct((B, 1), jnp.float32),
            jax.ShapeDtypeStruct((B, NC), jnp.float32),
        ),
    )(drug_emb, conf_embeddings, Wq, bq, Wk, bk, Wv, bv, Wh1, bh1, Wh2, bh2)
    return (logits2.squeeze(-1), attn)


# speedup vs baseline: 5.9498x; 1.9362x over previous
import functools

import jax
import jax.numpy as jnp
from jax import lax
from jax.experimental import pallas as pl
from jax.experimental.pallas import tpu as pltpu
from jax.experimental.pallas import tpu_sc as plsc

ND = 50000; ED = 800000; NP_ = 10000; EP = 160000; NC = 4; B = 256; DD = 78; DP = 128; H = 128

SC_CORES = 2
SC_TILES = 16

NDP = 50176
NPP = 10112
NCP = NC * NPP
ED_ROWS = 6400
EP_ROWS = 1280


def _deg_body(dst_drug, dst_conf, deg_drug, deg_conf, acc, idxbuf, ones_buf, zbuf):
    c = lax.axis_index("c")
    s = lax.axis_index("s")

    for k in range(8):
        ones_buf[pl.ds(k * 16, 16)] = jnp.full((16,), 1.0, jnp.float32)

    def _zero(i, _):
        zbuf[pl.ds(i * 16, 16)] = jnp.zeros((16,), jnp.float32)
        return 0

    lax.fori_loop(0, 3136 // 16, _zero, 0)

    pltpu.sync_copy(zbuf, acc.at[pl.ds(s * 3136, 3136)])
    plsc.subcore_barrier()

    @pl.when(c == 0)
    def _drug():
        def chunk(i, _):
            row0 = s * 400 + i * 16
            pltpu.sync_copy(dst_drug.at[pl.ds(row0, 16)], idxbuf)
            for j in range(16):
                pltpu.sync_copy(ones_buf, acc.at[idxbuf.at[j]], add=True)
            return 0

        lax.fori_loop(0, 25, chunk, 0)

    @pl.when(c == 1)
    def _conf():
        conf = s // 4
        sub = s % 4
        off = conf * NPP

        def chunk(i, _):
            row0 = conf * EP_ROWS + sub * 320 + i * 16
            pltpu.sync_copy(dst_conf.at[pl.ds(row0, 16)], idxbuf)
            for j in range(16):
                for g in range(8):
                    v = idxbuf[j, pl.ds(g * 16, 16)]
                    idxbuf[j, pl.ds(g * 16, 16)] = v + off
                pltpu.sync_copy(ones_buf, acc.at[idxbuf.at[j]], add=True)
            return 0

        lax.fori_loop(0, 20, chunk, 0)

    plsc.subcore_barrier()

    @pl.when(c == 0)
    def _out_drug():
        pltpu.sync_copy(acc.at[pl.ds(s * 3136, 3136)], zbuf)
        pltpu.sync_copy(zbuf, deg_drug.at[pl.ds(s * 3136, 3136)])

    @pl.when(c == 1)
    def _out_conf():
        pltpu.sync_copy(acc.at[pl.ds(s * 2528, 2528)], zbuf.at[pl.ds(0, 2528)])
        pltpu.sync_copy(zbuf.at[pl.ds(0, 2528)], deg_conf.at[pl.ds(s * 2528, 2528)])


_deg_kernel = pl.kernel(
    _deg_body,
    out_type=(
        jax.ShapeDtypeStruct((NDP,), jnp.float32),
        jax.ShapeDtypeStruct((NCP,), jnp.float32),
    ),
    mesh=plsc.VectorSubcoreMesh(
        core_axis_name="c", subcore_axis_name="s",
        num_cores=SC_CORES, num_subcores=SC_TILES),
    scratch_types=(
        pltpu.VMEM_SHARED((NDP,), jnp.float32),
        pltpu.VMEM((16, 128), jnp.int32),
        pltpu.VMEM((128,), jnp.float32),
        pltpu.VMEM((3136,), jnp.float32),
    ),
)


def _drug_agg_body(z0, z1, z2, z3, src2d, dst2d, out0, out1, out2, out3,
                   acc, sbuf, dbuf, rowbuf, tbuf):
    c = lax.axis_index("c")
    s = lax.axis_index("s")

    def _zero_tbuf(i, _):
        for g in range(2):
            tbuf[i, pl.ds(g * 16, 16)] = jnp.zeros((16,), jnp.float32)
        return 0

    def _run(zq, outq):
        lax.fori_loop(0, 448, _zero_tbuf, 0)
        for k in range(7):
            pltpu.sync_copy(tbuf, acc.at[pl.ds(s * 3136 + k * 448, 448)])
        plsc.subcore_barrier()

        def _chunk(i, _):
            row0 = s * 400 + i * 16
            pltpu.sync_copy(src2d.at[pl.ds(row0, 16)], sbuf)
            pltpu.sync_copy(dst2d.at[pl.ds(row0, 16)], dbuf)
            for j in range(16):
                pltpu.sync_copy(zq.at[sbuf.at[j]], rowbuf)
                pltpu.sync_copy(rowbuf, acc.at[dbuf.at[j]], add=True)
            return 0

        lax.fori_loop(0, 25, _chunk, 0)
        plsc.subcore_barrier()
        for k in range(7):
            pltpu.sync_copy(acc.at[pl.ds(s * 3136 + k * 448, 448)], tbuf)
            pltpu.sync_copy(tbuf, outq.at[pl.ds(s * 3136 + k * 448, 448)])

    for p in range(2):
        @pl.when(c == 0)
        def _qa():
            _run((z0, z2)[p], (out0, out2)[p])

        @pl.when(c == 1)
        def _qb():
            _run((z1, z3)[p], (out1, out3)[p])


_QT = jax.ShapeDtypeStruct((NDP, 32), jnp.float32)
_drug_agg = pl.kernel(
    _drug_agg_body,
    out_type=(_QT, _QT, _QT, _QT),
    mesh=plsc.VectorSubcoreMesh(
        core_axis_name="c", subcore_axis_name="s",
        num_cores=SC_CORES, num_subcores=SC_TILES),
    scratch_types=(
        pltpu.VMEM_SHARED((NDP, 32), jnp.float32),
        pltpu.VMEM((16, 128), jnp.int32),
        pltpu.VMEM((16, 128), jnp.int32),
        pltpu.VMEM((128, 32), jnp.float32),
        pltpu.VMEM((448, 32), jnp.float32),
    ),
    compiler_params=pltpu.CompilerParams(use_tc_tiling_on_sc=False),
)


def _prot_agg_body(z2d, src2d, dst2d, out, acc, sbuf, dbuf, rowbuf, tbuf):
    c = lax.axis_index("c")
    s = lax.axis_index("s")

    def _zero_tbuf(i, _):
        for g in range(8):
            tbuf[i, pl.ds(g * 16, 16)] = jnp.zeros((16,), jnp.float32)
        return 0

    for p in range(2):
        conf = c * 2 + p
        lax.fori_loop(0, 112, _zero_tbuf, 0)
        for k in range(5):
            pltpu.sync_copy(tbuf, acc.at[pl.ds(s * 632 + k * 112, 112)])
        pltpu.sync_copy(tbuf.at[pl.ds(0, 72)], acc.at[pl.ds(s * 632 + 560, 72)])
        plsc.subcore_barrier()

        def _chunk(i, _):
            row0 = conf * EP_ROWS + s * 80 + i * 16
            pltpu.sync_copy(src2d.at[pl.ds(row0, 16)], sbuf)
            pltpu.sync_copy(dst2d.at[pl.ds(row0, 16)], dbuf)
            for j in range(16):
                pltpu.sync_copy(z2d.at[sbuf.at[j]], rowbuf)
                pltpu.sync_copy(rowbuf, acc.at[dbuf.at[j]], add=True)
            return 0

        lax.fori_loop(0, 5, _chunk, 0)
        plsc.subcore_barrier()

        for k in range(5):
            pltpu.sync_copy(acc.at[pl.ds(s * 632 + k * 112, 112)], tbuf)
            pltpu.sync_copy(tbuf, out.at[pl.ds(conf * NPP + s * 632 + k * 112, 112)])
        pltpu.sync_copy(acc.at[pl.ds(s * 632 + 560, 72)], tbuf.at[pl.ds(0, 72)])
        pltpu.sync_copy(tbuf.at[pl.ds(0, 72)], out.at[pl.ds(conf * NPP + s * 632 + 560, 72)])


_prot_agg = pl.kernel(
    _prot_agg_body,
    out_type=jax.ShapeDtypeStruct((NCP, H), jnp.float32),
    mesh=plsc.VectorSubcoreMesh(
        core_axis_name="c", subcore_axis_name="s",
        num_cores=SC_CORES, num_subcores=SC_TILES),
    scratch_types=(
        pltpu.VMEM_SHARED((NPP, H), jnp.float32),
        pltpu.VMEM((16, 128), jnp.int32),
        pltpu.VMEM((16, 128), jnp.int32),
        pltpu.VMEM((128, H), jnp.float32),
        pltpu.VMEM((112, 128), jnp.float32),
    ),
)


def _compute_degrees(dst_drug2d, dst_conf2d):
    return _deg_kernel(dst_drug2d, dst_conf2d)


def _pool(x, seg, num_graphs):
    s = jax.ops.segment_sum(x, seg, num_segments=num_graphs)
    c = jax.ops.segment_sum(jnp.ones((x.shape[0], 1), x.dtype), seg, num_segments=num_graphs)
    return s / jnp.maximum(c, 1.0)


def _tail_body(drug_ref, conf_ref, Wq_ref, bq_ref, Wk_ref, bk_ref, Wv_ref, bv_ref,
               Wh1_ref, bh1_ref, Wh2_ref, bh2_ref, logits_ref, attn_ref):
    drug = drug_ref[...]
    q = drug @ Wq_ref[...] + bq_ref[...]
    scores = []
    vals = []
    for c in range(NC):
        kc = conf_ref[c] @ Wk_ref[...] + bk_ref[...]
        vc = conf_ref[c] @ Wv_ref[...] + bv_ref[...]
        scores.append(jnp.sum(q * kc, axis=1) / (H ** 0.5))
        vals.append(vc)
    sc = jnp.stack(scores, axis=1)
    m = jnp.max(sc, axis=1, keepdims=True)
    e = jnp.exp(sc - m)
    attn = e / jnp.sum(e, axis=1, keepdims=True)
    attended = sum(vals[c] * attn[:, c:c + 1] for c in range(NC))
    h1 = jnp.maximum(drug @ Wh1_ref[:H] + attended @ Wh1_ref[H:] + bh1_ref[...], 0.0)
    logits_ref[...] = h1 @ Wh2_ref[...] + bh2_ref[...]
    attn_ref[...] = attn


def kernel(drug_x, drug_edge_index, drug_batch_ids, conf_x, conf_edge_index, conf_batch_ids,
           Wd0, bd0, Wd1, bd1, Wd2, bd2, Wp0, bp0, Wp1, bp1, Wp2, bp2,
           Wq, bq, Wk, bk, Wv, bv, Wh1, bh1, Wh2, bh2):
    src2d = jnp.pad(drug_edge_index[0], (0, ED_ROWS * 128 - ED),
                    constant_values=0).reshape(ED_ROWS, 128).astype(jnp.int32)
    dst2d = jnp.pad(drug_edge_index[1], (0, ED_ROWS * 128 - ED),
                    constant_values=NDP - 1).reshape(ED_ROWS, 128).astype(jnp.int32)
    psrc = conf_edge_index[:, 0, :] + (jnp.arange(NC, dtype=jnp.int32) * NPP)[:, None]
    psrc2d = jnp.pad(psrc, ((0, 0), (0, EP_ROWS * 128 - EP)),
                     constant_values=0).reshape(NC * EP_ROWS, 128).astype(jnp.int32)
    pdst2d = jnp.pad(conf_edge_index[:, 1, :], ((0, 0), (0, EP_ROWS * 128 - EP)),
                     constant_values=NPP - 1).reshape(NC * EP_ROWS, 128).astype(jnp.int32)

    deg_drug, deg_conf = _compute_degrees(dst2d, pdst2d)
    dis_d = lax.rsqrt(deg_drug + 1.0)[:, None]
    dis_p = lax.rsqrt(deg_conf + 1.0)[:, None]

    h = jnp.pad(drug_x, ((0, NDP - ND), (0, 0)))
    for (W, b) in ((Wd0, bd0), (Wd1, bd1), (Wd2, bd2)):
        z = (h @ W) * dis_d
        s0, s1, s2, s3 = _drug_agg(z[:, 0:32], z[:, 32:64], z[:, 64:96], z[:, 96:128],
                                   src2d, dst2d)
        sagg = jnp.concatenate([s0, s1, s2, s3], axis=1)
        h = jax.nn.relu(dis_d * (sagg + z) + b)
    drug_emb = _pool(h[:ND], drug_batch_ids, B)

    hp = jnp.pad(conf_x, ((0, 0), (0, NPP - NP_), (0, 0))).reshape(NCP, DP)
    for (W, b) in ((Wp0, bp0), (Wp1, bp1), (Wp2, bp2)):
        z = (hp @ W) * dis_p
        sagg = _prot_agg(z, psrc2d, pdst2d)
        hp = jax.nn.relu(dis_p * (sagg + z) + b)
    hp4 = hp.reshape(NC, NPP, H)[:, :NP_]
    conf_embs = [_pool(hp4[i], conf_batch_ids, B) for i in range(NC)]
    conf_embeddings = jnp.stack(conf_embs, axis=0)

    logits2, attn = pl.pallas_call(
        _tail_body,
        out_shape=(
            jax.ShapeDtypeStruct((B, 1), jnp.float32),
            jax.ShapeDtypeStruct((B, NC), jnp.float32),
        ),
    )(drug_emb, conf_embeddings, Wq, bq, Wk, bk, Wv, bv, Wh1, bh1, Wh2, bh2)
    return (logits2.squeeze(-1), attn)


# speedup vs baseline: 7.3271x; 1.2315x over previous
import functools

import jax
import jax.numpy as jnp
from jax import lax
from jax.experimental import pallas as pl
from jax.experimental.pallas import tpu as pltpu
from jax.experimental.pallas import tpu_sc as plsc

ND = 50000; ED = 800000; NP_ = 10000; EP = 160000; NC = 4; B = 256; DD = 78; DP = 128; H = 128

SC_CORES = 2
SC_TILES = 16

NDP = 50176
NPP = 10240
NCP = NC * NPP
ED_ROWS = 6400
EP_ROWS = 1280


def _deg_body(dst_drug, dst_conf, deg_drug, deg_conf, acc, idxbuf, ones_buf, zbuf):
    c = lax.axis_index("c")
    s = lax.axis_index("s")

    for k in range(8):
        ones_buf[pl.ds(k * 16, 16)] = jnp.full((16,), 1.0, jnp.float32)

    def _zero(i, _):
        zbuf[pl.ds(i * 16, 16)] = jnp.zeros((16,), jnp.float32)
        return 0

    lax.fori_loop(0, 3136 // 16, _zero, 0)

    pltpu.sync_copy(zbuf, acc.at[pl.ds(s * 3136, 3136)])
    plsc.subcore_barrier()

    @pl.when(c == 0)
    def _drug():
        def chunk(i, _):
            row0 = s * 400 + i * 16
            pltpu.sync_copy(dst_drug.at[pl.ds(row0, 16)], idxbuf)
            for j in range(16):
                pltpu.sync_copy(ones_buf, acc.at[idxbuf.at[j]], add=True)
            return 0

        lax.fori_loop(0, 25, chunk, 0)

    @pl.when(c == 1)
    def _conf():
        conf = s // 4
        sub = s % 4
        off = conf * NPP

        def chunk(i, _):
            row0 = conf * EP_ROWS + sub * 320 + i * 16
            pltpu.sync_copy(dst_conf.at[pl.ds(row0, 16)], idxbuf)
            for j in range(16):
                for g in range(8):
                    v = idxbuf[j, pl.ds(g * 16, 16)]
                    idxbuf[j, pl.ds(g * 16, 16)] = v + off
                pltpu.sync_copy(ones_buf, acc.at[idxbuf.at[j]], add=True)
            return 0

        lax.fori_loop(0, 20, chunk, 0)

    plsc.subcore_barrier()

    @pl.when(c == 0)
    def _out_drug():
        pltpu.sync_copy(acc.at[pl.ds(s * 3136, 3136)], zbuf)
        pltpu.sync_copy(zbuf, deg_drug.at[pl.ds(s * 3136, 3136)])

    @pl.when(c == 1)
    def _out_conf():
        pltpu.sync_copy(acc.at[pl.ds(s * 2560, 2560)], zbuf.at[pl.ds(0, 2560)])
        pltpu.sync_copy(zbuf.at[pl.ds(0, 2560)], deg_conf.at[pl.ds(s * 2560, 2560)])


_deg_kernel = pl.kernel(
    _deg_body,
    out_type=(
        jax.ShapeDtypeStruct((NDP,), jnp.float32),
        jax.ShapeDtypeStruct((NCP,), jnp.float32),
    ),
    mesh=plsc.VectorSubcoreMesh(
        core_axis_name="c", subcore_axis_name="s",
        num_cores=SC_CORES, num_subcores=SC_TILES),
    scratch_types=(
        pltpu.VMEM_SHARED((NDP,), jnp.float32),
        pltpu.VMEM((16, 128), jnp.int32),
        pltpu.VMEM((128,), jnp.float32),
        pltpu.VMEM((3136,), jnp.float32),
    ),
)


def _make_agg(nodes, n_chunks, wo_chunk, wo_n):
    rows_per_tile = wo_chunk * wo_n

    def body(z0, z1, z2, z3, src2d, dst2d, o0, o1, o2, o3,
             acc, sbuf, dbuf, bufs, tbuf, gsem, ssem):
        c = lax.axis_index("c")
        s = lax.axis_index("s")

        def _zero_tbuf(i, _):
            for g in range(2):
                tbuf[i, pl.ds(g * 16, 16)] = jnp.zeros((16,), jnp.float32)
            return 0

        def _run(zq, outq):
            lax.fori_loop(0, wo_chunk, _zero_tbuf, 0)
            for k in range(wo_n):
                pltpu.sync_copy(tbuf.at[pl.ds(0, wo_chunk)],
                                acc.at[pl.ds(s * rows_per_tile + k * wo_chunk, wo_chunk)])
            plsc.subcore_barrier()

            def _chunk(i, _):
                row0 = s * (n_chunks * 16) + i * 16
                pltpu.sync_copy(src2d.at[pl.ds(row0, 16)], sbuf)
                pltpu.sync_copy(dst2d.at[pl.ds(row0, 16)], dbuf)
                gd = {}
                sd = {}

                def fire_g(g):
                    for t in range(2):
                        j = g * 2 + t
                        gd[j] = pltpu.async_copy(
                            zq.at[sbuf.at[j]], bufs.at[(g % 2) * 2 + t], gsem)

                def fire_s(g):
                    for t in range(2):
                        j = g * 2 + t
                        sd[j] = pltpu.async_copy(
                            bufs.at[(g % 2) * 2 + t], acc.at[dbuf.at[j]], ssem,
                            add=True)

                fire_g(0)
                for g in range(8):
                    for t in range(2):
                        gd[g * 2 + t].wait()
                    fire_s(g)
                    if g >= 1:
                        for t in range(2):
                            sd[(g - 1) * 2 + t].wait()
                    if g + 1 < 8:
                        fire_g(g + 1)
                for t in range(2):
                    sd[14 + t].wait()
                return 0

            lax.fori_loop(0, n_chunks, _chunk, 0)
            plsc.subcore_barrier()
            for k in range(wo_n):
                pltpu.sync_copy(acc.at[pl.ds(s * rows_per_tile + k * wo_chunk, wo_chunk)],
                                tbuf.at[pl.ds(0, wo_chunk)])
                pltpu.sync_copy(tbuf.at[pl.ds(0, wo_chunk)],
                                outq.at[pl.ds(s * rows_per_tile + k * wo_chunk, wo_chunk)])

        for p in range(2):
            @pl.when(c == 0)
            def _qa():
                _run((z0, z2)[p], (o0, o2)[p])

            @pl.when(c == 1)
            def _qb():
                _run((z1, z3)[p], (o1, o3)[p])

    qt = jax.ShapeDtypeStruct((nodes, 32), jnp.float32)
    return pl.kernel(
        body,
        out_type=(qt, qt, qt, qt),
        mesh=plsc.VectorSubcoreMesh(
            core_axis_name="c", subcore_axis_name="s",
            num_cores=SC_CORES, num_subcores=SC_TILES),
        scratch_types=(
            pltpu.VMEM_SHARED((nodes, 32), jnp.float32),
            pltpu.VMEM((16, 128), jnp.int32),
            pltpu.VMEM((16, 128), jnp.int32),
            pltpu.VMEM((4, 128, 32), jnp.float32),
            pltpu.VMEM((160, 32), jnp.float32),
            pltpu.SemaphoreType.DMA,
            pltpu.SemaphoreType.DMA,
        ),
        compiler_params=pltpu.CompilerParams(use_tc_tiling_on_sc=False),
    )


_drug_agg = _make_agg(NDP, 25, 112, 28)
_prot_agg = _make_agg(NCP, 20, 160, 16)


def _compute_degrees(dst_drug2d, dst_conf2d):
    return _deg_kernel(dst_drug2d, dst_conf2d)


def _pool(x, seg, num_graphs):
    s = jax.ops.segment_sum(x, seg, num_segments=num_graphs)
    c = jax.ops.segment_sum(jnp.ones((x.shape[0], 1), x.dtype), seg, num_segments=num_graphs)
    return s / jnp.maximum(c, 1.0)


def _tail_body(drug_ref, conf_ref, Wq_ref, bq_ref, Wk_ref, bk_ref, Wv_ref, bv_ref,
               Wh1_ref, bh1_ref, Wh2_ref, bh2_ref, logits_ref, attn_ref):
    drug = drug_ref[...]
    q = drug @ Wq_ref[...] + bq_ref[...]
    scores = []
    vals = []
    for c in range(NC):
        kc = conf_ref[c] @ Wk_ref[...] + bk_ref[...]
        vc = conf_ref[c] @ Wv_ref[...] + bv_ref[...]
        scores.append(jnp.sum(q * kc, axis=1) / (H ** 0.5))
        vals.append(vc)
    sc = jnp.stack(scores, axis=1)
    m = jnp.max(sc, axis=1, keepdims=True)
    e = jnp.exp(sc - m)
    attn = e / jnp.sum(e, axis=1, keepdims=True)
    attended = sum(vals[c] * attn[:, c:c + 1] for c in range(NC))
    h1 = jnp.maximum(drug @ Wh1_ref[:H] + attended @ Wh1_ref[H:] + bh1_ref[...], 0.0)
    logits_ref[...] = h1 @ Wh2_ref[...] + bh2_ref[...]
    attn_ref[...] = attn


def kernel(drug_x, drug_edge_index, drug_batch_ids, conf_x, conf_edge_index, conf_batch_ids,
           Wd0, bd0, Wd1, bd1, Wd2, bd2, Wp0, bp0, Wp1, bp1, Wp2, bp2,
           Wq, bq, Wk, bk, Wv, bv, Wh1, bh1, Wh2, bh2):
    src2d = jnp.pad(drug_edge_index[0], (0, ED_ROWS * 128 - ED),
                    constant_values=0).reshape(ED_ROWS, 128).astype(jnp.int32)
    dst2d = jnp.pad(drug_edge_index[1], (0, ED_ROWS * 128 - ED),
                    constant_values=NDP - 1).reshape(ED_ROWS, 128).astype(jnp.int32)
    psrc = conf_edge_index[:, 0, :] + (jnp.arange(NC, dtype=jnp.int32) * NPP)[:, None]
    psrc2d = jnp.pad(psrc, ((0, 0), (0, EP_ROWS * 128 - EP)),
                     constant_values=0).reshape(NC * EP_ROWS, 128).astype(jnp.int32)
    pdst_loc2d = jnp.pad(conf_edge_index[:, 1, :], ((0, 0), (0, EP_ROWS * 128 - EP)),
                         constant_values=NPP - 1).reshape(NC * EP_ROWS, 128).astype(jnp.int32)
    pdst = conf_edge_index[:, 1, :] + (jnp.arange(NC, dtype=jnp.int32) * NPP)[:, None]
    pdst2d = jnp.pad(pdst, ((0, 0), (0, EP_ROWS * 128 - EP)),
                     constant_values=NCP - 1).reshape(NC * EP_ROWS, 128).astype(jnp.int32)

    deg_drug, deg_conf = _compute_degrees(dst2d, pdst_loc2d)
    dis_d = lax.rsqrt(deg_drug + 1.0)[:, None]
    dis_p = lax.rsqrt(deg_conf + 1.0)[:, None]

    h = jnp.pad(drug_x, ((0, NDP - ND), (0, 0)))
    for (W, b) in ((Wd0, bd0), (Wd1, bd1), (Wd2, bd2)):
        z = (h @ W) * dis_d
        s0, s1, s2, s3 = _drug_agg(z[:, 0:32], z[:, 32:64], z[:, 64:96], z[:, 96:128],
                                   src2d, dst2d)
        sagg = jnp.concatenate([s0, s1, s2, s3], axis=1)
        h = jax.nn.relu(dis_d * (sagg + z) + b)
    drug_emb = _pool(h[:ND], drug_batch_ids, B)

    hp = jnp.pad(conf_x, ((0, 0), (0, NPP - NP_), (0, 0))).reshape(NCP, DP)
    for (W, b) in ((Wp0, bp0), (Wp1, bp1), (Wp2, bp2)):
        z = (hp @ W) * dis_p
        s0, s1, s2, s3 = _prot_agg(z[:, 0:32], z[:, 32:64], z[:, 64:96], z[:, 96:128],
                                   psrc2d, pdst2d)
        sagg = jnp.concatenate([s0, s1, s2, s3], axis=1)
        hp = jax.nn.relu(dis_p * (sagg + z) + b)
    hp4 = hp.reshape(NC, NPP, H)[:, :NP_]
    conf_embs = [_pool(hp4[i], conf_batch_ids, B) for i in range(NC)]
    conf_embeddings = jnp.stack(conf_embs, axis=0)

    logits2, attn = pl.pallas_call(
        _tail_body,
        out_shape=(
            jax.ShapeDtypeStruct((B, 1), jnp.float32),
            jax.ShapeDtypeStruct((B, NC), jnp.float32),
        ),
    )(drug_emb, conf_embeddings, Wq, bq, Wk, bk, Wv, bv, Wh1, bh1, Wh2, bh2)
    return (logits2.squeeze(-1), attn)


# speedup vs baseline: 7.6730x; 1.0472x over previous
import functools

import jax
import jax.numpy as jnp
from jax import lax
from jax.experimental import pallas as pl
from jax.experimental.pallas import tpu as pltpu
from jax.experimental.pallas import tpu_sc as plsc

ND = 50000; ED = 800000; NP_ = 10000; EP = 160000; NC = 4; B = 256; DD = 78; DP = 128; H = 128

SC_CORES = 2
SC_TILES = 16

NDP = 50176
NPP = 10240
NCP = NC * NPP
ED_ROWS = 6400
EP_ROWS = 1280


def _deg_body(dst_drug, dst_conf, deg_drug, deg_conf, acc, idxbuf, ones_buf, zbuf):
    c = lax.axis_index("c")
    s = lax.axis_index("s")

    for k in range(8):
        ones_buf[pl.ds(k * 16, 16)] = jnp.full((16,), 1.0, jnp.float32)

    def _zero(i, _):
        zbuf[pl.ds(i * 16, 16)] = jnp.zeros((16,), jnp.float32)
        return 0

    lax.fori_loop(0, 3136 // 16, _zero, 0)

    pltpu.sync_copy(zbuf, acc.at[pl.ds(s * 3136, 3136)])
    plsc.subcore_barrier()

    @pl.when(c == 0)
    def _drug():
        def chunk(i, _):
            row0 = s * 400 + i * 16
            pltpu.sync_copy(dst_drug.at[pl.ds(row0, 16)], idxbuf)
            for j in range(16):
                pltpu.sync_copy(ones_buf, acc.at[idxbuf.at[j]], add=True)
            return 0

        lax.fori_loop(0, 25, chunk, 0)

    @pl.when(c == 1)
    def _conf():
        conf = s // 4
        sub = s % 4
        off = conf * NPP

        def chunk(i, _):
            row0 = conf * EP_ROWS + sub * 320 + i * 16
            pltpu.sync_copy(dst_conf.at[pl.ds(row0, 16)], idxbuf)
            for j in range(16):
                for g in range(8):
                    v = idxbuf[j, pl.ds(g * 16, 16)]
                    idxbuf[j, pl.ds(g * 16, 16)] = v + off
                pltpu.sync_copy(ones_buf, acc.at[idxbuf.at[j]], add=True)
            return 0

        lax.fori_loop(0, 20, chunk, 0)

    plsc.subcore_barrier()

    @pl.when(c == 0)
    def _out_drug():
        pltpu.sync_copy(acc.at[pl.ds(s * 3136, 3136)], zbuf)
        pltpu.sync_copy(zbuf, deg_drug.at[pl.ds(s * 3136, 3136)])

    @pl.when(c == 1)
    def _out_conf():
        pltpu.sync_copy(acc.at[pl.ds(s * 2560, 2560)], zbuf.at[pl.ds(0, 2560)])
        pltpu.sync_copy(zbuf.at[pl.ds(0, 2560)], deg_conf.at[pl.ds(s * 2560, 2560)])


_deg_kernel = pl.kernel(
    _deg_body,
    out_type=(
        jax.ShapeDtypeStruct((NDP,), jnp.float32),
        jax.ShapeDtypeStruct((NCP,), jnp.float32),
    ),
    mesh=plsc.VectorSubcoreMesh(
        core_axis_name="c", subcore_axis_name="s",
        num_cores=SC_CORES, num_subcores=SC_TILES),
    scratch_types=(
        pltpu.VMEM_SHARED((NDP,), jnp.float32),
        pltpu.VMEM((16, 128), jnp.int32),
        pltpu.VMEM((128,), jnp.float32),
        pltpu.VMEM((3136,), jnp.float32),
    ),
)


def _make_agg(nodes, n_chunks, wo_chunk, wo_n):
    rows_per_tile = wo_chunk * wo_n

    def body(z0, z1, z2, z3, src2d, dst2d, o0, o1, o2, o3,
             acc, sbuf, dbuf, bufs, tbuf, gsem, ssem):
        c = lax.axis_index("c")
        s = lax.axis_index("s")

        def _zero_tbuf(i, _):
            for g in range(2):
                tbuf[i, pl.ds(g * 16, 16)] = jnp.zeros((16,), jnp.float32)
            return 0

        def _run(zq, outq):
            lax.fori_loop(0, wo_chunk, _zero_tbuf, 0)
            for k in range(wo_n):
                pltpu.sync_copy(tbuf.at[pl.ds(0, wo_chunk)],
                                acc.at[pl.ds(s * rows_per_tile + k * wo_chunk, wo_chunk)])
            plsc.subcore_barrier()

            def _chunk(i, _):
                row0 = s * (n_chunks * 16) + i * 16
                pltpu.sync_copy(src2d.at[pl.ds(row0, 16)], sbuf)
                pltpu.sync_copy(dst2d.at[pl.ds(row0, 16)], dbuf)
                gd = {}
                sd = {}

                def fire_g(g):
                    for t in range(2):
                        j = g * 2 + t
                        gd[j] = pltpu.async_copy(
                            zq.at[sbuf.at[j]], bufs.at[(g % 2) * 2 + t], gsem)

                def fire_s(g):
                    for t in range(2):
                        j = g * 2 + t
                        sd[j] = pltpu.async_copy(
                            bufs.at[(g % 2) * 2 + t], acc.at[dbuf.at[j]], ssem,
                            add=True)

                fire_g(0)
                for g in range(8):
                    for t in range(2):
                        gd[g * 2 + t].wait()
                    fire_s(g)
                    if g >= 1:
                        for t in range(2):
                            sd[(g - 1) * 2 + t].wait()
                    if g + 1 < 8:
                        fire_g(g + 1)
                for t in range(2):
                    sd[14 + t].wait()
                return 0

            lax.fori_loop(0, n_chunks, _chunk, 0)
            plsc.subcore_barrier()
            for k in range(wo_n):
                pltpu.sync_copy(acc.at[pl.ds(s * rows_per_tile + k * wo_chunk, wo_chunk)],
                                tbuf.at[pl.ds(0, wo_chunk)])
                pltpu.sync_copy(tbuf.at[pl.ds(0, wo_chunk)],
                                outq.at[pl.ds(s * rows_per_tile + k * wo_chunk, wo_chunk)])

        for p in range(2):
            @pl.when(c == 0)
            def _qa():
                _run((z0, z2)[p], (o0, o2)[p])

            @pl.when(c == 1)
            def _qb():
                _run((z1, z3)[p], (o1, o3)[p])

    qt = jax.ShapeDtypeStruct((nodes, 32), jnp.float32)
    return pl.kernel(
        body,
        out_type=(qt, qt, qt, qt),
        mesh=plsc.VectorSubcoreMesh(
            core_axis_name="c", subcore_axis_name="s",
            num_cores=SC_CORES, num_subcores=SC_TILES),
        scratch_types=(
            pltpu.VMEM_SHARED((nodes, 32), jnp.float32),
            pltpu.VMEM((16, 128), jnp.int32),
            pltpu.VMEM((16, 128), jnp.int32),
            pltpu.VMEM((4, 128, 32), jnp.float32),
            pltpu.VMEM((160, 32), jnp.float32),
            pltpu.SemaphoreType.DMA,
            pltpu.SemaphoreType.DMA,
        ),
        compiler_params=pltpu.CompilerParams(use_tc_tiling_on_sc=False),
    )


_drug_agg = _make_agg(NDP, 25, 112, 28)
_prot_agg = _make_agg(NCP, 20, 160, 16)


def _make_tc(n, bn):
    grid = (n // bn,)
    qspec = pl.BlockSpec((bn, 32), lambda i: (i, 0))
    fspec = pl.BlockSpec((bn, 128), lambda i: (i, 0))
    wspec = pl.BlockSpec((128, 128), lambda i: (0, 0))
    bspec = pl.BlockSpec((1, 128), lambda i: (0, 0))
    qt = jax.ShapeDtypeStruct((n, 32), jnp.float32)
    ft = jax.ShapeDtypeStruct((n, 128), jnp.float32)

    def pre_body(x_ref, W_ref, dis_ref, o0, o1, o2, o3):
        z = jnp.dot(x_ref[...], W_ref[...],
                    preferred_element_type=jnp.float32) * dis_ref[...]
        for q, o in enumerate((o0, o1, o2, o3)):
            o[...] = z[:, q * 32:(q + 1) * 32]

    pre = pl.pallas_call(pre_body, grid=grid,
                         in_specs=[fspec, wspec, fspec],
                         out_specs=(qspec,) * 4, out_shape=(qt,) * 4)

    def mid_body(s0, s1, s2, s3, z0, z1, z2, z3, dis_ref, b_ref, W_ref,
                 o0, o1, o2, o3):
        sagg = jnp.concatenate([s0[...], s1[...], s2[...], s3[...]], axis=1)
        z = jnp.concatenate([z0[...], z1[...], z2[...], z3[...]], axis=1)
        d = dis_ref[...]
        h = jnp.maximum(d * (sagg + z) + b_ref[...], 0.0)
        zn = jnp.dot(h, W_ref[...], preferred_element_type=jnp.float32) * d
        for q, o in enumerate((o0, o1, o2, o3)):
            o[...] = zn[:, q * 32:(q + 1) * 32]

    mid = pl.pallas_call(mid_body, grid=grid,
                         in_specs=[qspec] * 8 + [fspec, bspec, wspec],
                         out_specs=(qspec,) * 4, out_shape=(qt,) * 4)

    def fin_body(s0, s1, s2, s3, z0, z1, z2, z3, dis_ref, b_ref, h_out):
        sagg = jnp.concatenate([s0[...], s1[...], s2[...], s3[...]], axis=1)
        z = jnp.concatenate([z0[...], z1[...], z2[...], z3[...]], axis=1)
        h_out[...] = jnp.maximum(dis_ref[...] * (sagg + z) + b_ref[...], 0.0)

    fin = pl.pallas_call(fin_body, grid=grid,
                         in_specs=[qspec] * 8 + [fspec, bspec],
                         out_specs=fspec, out_shape=ft)
    return pre, mid, fin


_drug_pre, _drug_mid, _drug_fin = _make_tc(NDP, 3136)
_prot_pre, _prot_mid, _prot_fin = _make_tc(NCP, 2560)


def _pool_body(hd, hp, idsd, idsdc, idsp, idspc, dpool, ppool,
               acc, ibuf, ibufc, rowbuf, onesb, tbuf):
    c = lax.axis_index("c")
    s = lax.axis_index("s")

    def _ones(i, _):
        for g in range(8):
            onesb[i, pl.ds(g * 16, 16)] = jnp.full((16,), 1.0, jnp.float32)
        return 0

    def _zero(i, _):
        for g in range(8):
            tbuf[i, pl.ds(g * 16, 16)] = jnp.zeros((16,), jnp.float32)
        return 0

    lax.fori_loop(0, 128, _ones, 0)
    lax.fori_loop(0, 136, _zero, 0)
    pltpu.sync_copy(tbuf, acc.at[pl.ds(s * 136, 136)])
    plsc.subcore_barrier()

    @pl.when(c == 0)
    def _drug():
        nch = jnp.where(s == 0, 4, 3)
        base = jnp.where(s == 0, 0, 8 + s * 24)

        def ch(i, _):
            r0 = base + i * 8
            pltpu.sync_copy(idsd.at[pl.ds(r0, 8)], ibuf)
            pltpu.sync_copy(idsdc.at[pl.ds(r0, 8)], ibufc)
            for j in range(8):
                pltpu.sync_copy(hd.at[pl.ds((r0 + j) * 128, 128)], rowbuf)
                pltpu.sync_copy(rowbuf, acc.at[ibuf.at[j]], add=True)
                pltpu.sync_copy(onesb, acc.at[ibufc.at[j]], add=True)
            return 0

        lax.fori_loop(0, nch, ch, 0)

    @pl.when(c == 1)
    def _prot():
        nch = jnp.where(s < 8, 3, 2)
        base = jnp.where(s < 8, s * 24, 192 + (s - 8) * 16)

        def ch(i, _):
            r0 = base + i * 8
            pltpu.sync_copy(idsp.at[pl.ds(r0, 8)], ibuf)
            pltpu.sync_copy(idspc.at[pl.ds(r0, 8)], ibufc)
            for j in range(8):
                pltpu.sync_copy(hp.at[pl.ds((r0 + j) * 128, 128)], rowbuf)
                pltpu.sync_copy(rowbuf, acc.at[ibuf.at[j]], add=True)
                pltpu.sync_copy(onesb, acc.at[ibufc.at[j]], add=True)
            return 0

        lax.fori_loop(0, nch, ch, 0)

    plsc.subcore_barrier()

    @pl.when(c == 0)
    def _out_d():
        pltpu.sync_copy(acc.at[pl.ds(s * 40, 40)], tbuf.at[pl.ds(0, 40)])
        pltpu.sync_copy(tbuf.at[pl.ds(0, 40)], dpool.at[pl.ds(s * 40, 40)])

    @pl.when(c == 1)
    def _out_p():
        pltpu.sync_copy(acc.at[pl.ds(s * 136, 136)], tbuf)
        pltpu.sync_copy(tbuf, ppool.at[pl.ds(s * 136, 136)])


_pool_kernel = pl.kernel(
    _pool_body,
    out_type=(
        jax.ShapeDtypeStruct((640, 128), jnp.float32),
        jax.ShapeDtypeStruct((2176, 128), jnp.float32),
    ),
    mesh=plsc.VectorSubcoreMesh(
        core_axis_name="c", subcore_axis_name="s",
        num_cores=SC_CORES, num_subcores=SC_TILES),
    scratch_types=(
        pltpu.VMEM_SHARED((2176, 128), jnp.float32),
        pltpu.VMEM((8, 128), jnp.int32),
        pltpu.VMEM((8, 128), jnp.int32),
        pltpu.VMEM((128, 128), jnp.float32),
        pltpu.VMEM((128, 128), jnp.float32),
        pltpu.VMEM((136, 128), jnp.float32),
    ),
)


def _compute_degrees(dst_drug2d, dst_conf2d):
    return _deg_kernel(dst_drug2d, dst_conf2d)


def _tail_body(dpool_ref, ppool_ref, Wq_ref, bq_ref, Wk_ref, bk_ref, Wv_ref, bv_ref,
               Wh1_ref, bh1_ref, Wh2_ref, bh2_ref, logits_ref, attn_ref):
    drug = dpool_ref[0:256] / jnp.maximum(dpool_ref[320:576], 1.0)
    q = drug @ Wq_ref[...] + bq_ref[...]
    scores = []
    vals = []
    for c in range(NC):
        pc = (ppool_ref[c * 264:c * 264 + 256]
              / jnp.maximum(ppool_ref[1088 + c * 264:1088 + c * 264 + 256], 1.0))
        kc = pc @ Wk_ref[...] + bk_ref[...]
        vc = pc @ Wv_ref[...] + bv_ref[...]
        scores.append(jnp.sum(q * kc, axis=1) / (H ** 0.5))
        vals.append(vc)
    sc = jnp.stack(scores, axis=1)
    m = jnp.max(sc, axis=1, keepdims=True)
    e = jnp.exp(sc - m)
    attn = e / jnp.sum(e, axis=1, keepdims=True)
    attended = sum(vals[c] * attn[:, c:c + 1] for c in range(NC))
    h1 = jnp.maximum(drug @ Wh1_ref[:H] + attended @ Wh1_ref[H:] + bh1_ref[...], 0.0)
    logits_ref[...] = h1 @ Wh2_ref[...] + bh2_ref[...]
    attn_ref[...] = attn


def kernel(drug_x, drug_edge_index, drug_batch_ids, conf_x, conf_edge_index, conf_batch_ids,
           Wd0, bd0, Wd1, bd1, Wd2, bd2, Wp0, bp0, Wp1, bp1, Wp2, bp2,
           Wq, bq, Wk, bk, Wv, bv, Wh1, bh1, Wh2, bh2):
    src2d = jnp.pad(drug_edge_index[0], (0, ED_ROWS * 128 - ED),
                    constant_values=0).reshape(ED_ROWS, 128).astype(jnp.int32)
    dst2d = jnp.pad(drug_edge_index[1], (0, ED_ROWS * 128 - ED),
                    constant_values=NDP - 1).reshape(ED_ROWS, 128).astype(jnp.int32)
    psrc = conf_edge_index[:, 0, :] + (jnp.arange(NC, dtype=jnp.int32) * NPP)[:, None]
    psrc2d = jnp.pad(psrc, ((0, 0), (0, EP_ROWS * 128 - EP)),
                     constant_values=0).reshape(NC * EP_ROWS, 128).astype(jnp.int32)
    pdst_loc2d = jnp.pad(conf_edge_index[:, 1, :], ((0, 0), (0, EP_ROWS * 128 - EP)),
                         constant_values=NPP - 1).reshape(NC * EP_ROWS, 128).astype(jnp.int32)
    pdst = conf_edge_index[:, 1, :] + (jnp.arange(NC, dtype=jnp.int32) * NPP)[:, None]
    pdst2d = jnp.pad(pdst, ((0, 0), (0, EP_ROWS * 128 - EP)),
                     constant_values=NCP - 1).reshape(NC * EP_ROWS, 128).astype(jnp.int32)

    idsd2d = jnp.concatenate(
        [drug_batch_ids.astype(jnp.int32),
         jnp.full((NDP - ND,), 256, jnp.int32)]).reshape(NDP // 128, 128)
    idsdc2d = idsd2d + 320
    pb = conf_batch_ids.astype(jnp.int32)
    idsp2d = jnp.concatenate(
        [jnp.concatenate([pb + c * 264, jnp.full((NPP - NP_,), 1056, jnp.int32)])
         for c in range(NC)]).reshape(NCP // 128, 128)
    idspc2d = idsp2d + 1088

    deg_drug, deg_conf = _compute_degrees(dst2d, pdst_loc2d)
    dis_d2 = jnp.broadcast_to(lax.rsqrt(deg_drug + 1.0)[:, None], (NDP, 128))
    dis_p2 = jnp.broadcast_to(lax.rsqrt(deg_conf + 1.0)[:, None], (NCP, 128))

    x_pad = jnp.pad(drug_x, ((0, NDP - ND), (0, 128 - DD)))
    Wd0p = jnp.pad(Wd0, ((0, 128 - DD), (0, 0)))
    zq = _drug_pre(x_pad, Wd0p, dis_d2)
    for (b_prev, W) in ((bd0, Wd1), (bd1, Wd2)):
        sq = _drug_agg(*zq, src2d, dst2d)
        zq = _drug_mid(*sq, *zq, dis_d2, b_prev[None, :], W)
    sq = _drug_agg(*zq, src2d, dst2d)
    hd = _drug_fin(*sq, *zq, dis_d2, bd2[None, :])

    hp0 = jnp.pad(conf_x, ((0, 0), (0, NPP - NP_), (0, 0))).reshape(NCP, DP)
    zq = _prot_pre(hp0, Wp0, dis_p2)
    for (b_prev, W) in ((bp0, Wp1), (bp1, Wp2)):
        sq = _prot_agg(*zq, psrc2d, pdst2d)
        zq = _prot_mid(*sq, *zq, dis_p2, b_prev[None, :], W)
    sq = _prot_agg(*zq, psrc2d, pdst2d)
    hp = _prot_fin(*sq, *zq, dis_p2, bp2[None, :])

    dpool, ppool = _pool_kernel(hd, hp, idsd2d, idsdc2d, idsp2d, idspc2d)
    logits2, attn = pl.pallas_call(
        _tail_body,
        out_shape=(
            jax.ShapeDtypeStruct((B, 1), jnp.float32),
            jax.ShapeDtypeStruct((B, NC), jnp.float32),
        ),
    )(dpool, ppool, Wq, bq, Wk, bk, Wv, bv, Wh1, bh1, Wh2, bh2)
    return (logits2.squeeze(-1), attn)


# speedup vs baseline: 8.7109x; 1.1353x over previous
import functools

import jax
import jax.numpy as jnp
from jax import lax
from jax.experimental import pallas as pl
from jax.experimental.pallas import tpu as pltpu
from jax.experimental.pallas import tpu_sc as plsc

ND = 50000; ED = 800000; NP_ = 10000; EP = 160000; NC = 4; B = 256; DD = 78; DP = 128; H = 128

SC_CORES = 2
SC_TILES = 16

NDP = 50176
NPP = 10240
NCP = NC * NPP
ED_ROWS = 6400
EP_ROWS = 1280


def _deg_body(dst_drug, dst_conf, deg_drug, deg_conf, acc, idxbuf, ones_buf, zbuf):
    c = lax.axis_index("c")
    s = lax.axis_index("s")

    for k in range(8):
        ones_buf[pl.ds(k * 16, 16)] = jnp.full((16,), 1.0, jnp.float32)

    def _zero(i, _):
        zbuf[pl.ds(i * 16, 16)] = jnp.zeros((16,), jnp.float32)
        return 0

    lax.fori_loop(0, 3136 // 16, _zero, 0)

    pltpu.sync_copy(zbuf, acc.at[pl.ds(s * 3136, 3136)])
    plsc.subcore_barrier()

    @pl.when(c == 0)
    def _drug():
        def chunk(i, _):
            row0 = s * 400 + i * 16
            pltpu.sync_copy(dst_drug.at[pl.ds(row0, 16)], idxbuf)
            for j in range(16):
                pltpu.sync_copy(ones_buf, acc.at[idxbuf.at[j]], add=True)
            return 0

        lax.fori_loop(0, 25, chunk, 0)

    @pl.when(c == 1)
    def _conf():
        conf = s // 4
        sub = s % 4
        off = conf * NPP

        def chunk(i, _):
            row0 = conf * EP_ROWS + sub * 320 + i * 16
            pltpu.sync_copy(dst_conf.at[pl.ds(row0, 16)], idxbuf)
            for j in range(16):
                for g in range(8):
                    v = idxbuf[j, pl.ds(g * 16, 16)]
                    idxbuf[j, pl.ds(g * 16, 16)] = v + off
                pltpu.sync_copy(ones_buf, acc.at[idxbuf.at[j]], add=True)
            return 0

        lax.fori_loop(0, 20, chunk, 0)

    plsc.subcore_barrier()

    @pl.when(c == 0)
    def _out_drug():
        pltpu.sync_copy(acc.at[pl.ds(s * 3136, 3136)], zbuf)
        pltpu.sync_copy(zbuf, deg_drug.at[pl.ds(s * 3136, 3136)])

    @pl.when(c == 1)
    def _out_conf():
        pltpu.sync_copy(acc.at[pl.ds(s * 2560, 2560)], zbuf.at[pl.ds(0, 2560)])
        pltpu.sync_copy(zbuf.at[pl.ds(0, 2560)], deg_conf.at[pl.ds(s * 2560, 2560)])


_deg_kernel = pl.kernel(
    _deg_body,
    out_type=(
        jax.ShapeDtypeStruct((NDP,), jnp.float32),
        jax.ShapeDtypeStruct((NCP,), jnp.float32),
    ),
    mesh=plsc.VectorSubcoreMesh(
        core_axis_name="c", subcore_axis_name="s",
        num_cores=SC_CORES, num_subcores=SC_TILES),
    scratch_types=(
        pltpu.VMEM_SHARED((NDP,), jnp.float32),
        pltpu.VMEM((16, 128), jnp.int32),
        pltpu.VMEM((128,), jnp.float32),
        pltpu.VMEM((3136,), jnp.float32),
    ),
)


def _make_agg(nodes, n_chunks, wo_chunk, wo_n):
    rows_per_tile = wo_chunk * wo_n
    ch = 40

    def body(z0, z1, z2, z3, src2d, dst2d, o0, o1, o2, o3,
             acc, sbuf, dbuf, bufs, tbuf, gsem, ssem):
        c = lax.axis_index("c")
        s = lax.axis_index("s")

        def _zero_tbuf(i, _):
            for g in range(2):
                tbuf[i, pl.ds(g * 16, 16)] = jnp.zeros((16,), jnp.float32)
            return 0

        def _run(zq, outq):
            lax.fori_loop(0, wo_chunk, _zero_tbuf, 0)
            for k in range(wo_n):
                pltpu.sync_copy(tbuf.at[pl.ds(0, wo_chunk)],
                                acc.at[pl.ds(s * rows_per_tile + k * wo_chunk, wo_chunk)])
            plsc.subcore_barrier()

            def _chunk(i, _):
                row0 = s * (n_chunks * ch) + i * ch
                pltpu.sync_copy(src2d.at[pl.ds(row0, ch)], sbuf)
                pltpu.sync_copy(dst2d.at[pl.ds(row0, ch)], dbuf)
                gd = {}
                sd = {}

                def fg(j):
                    gd[j] = pltpu.async_copy(zq.at[sbuf.at[j]], bufs.at[j % 4], gsem)

                def fs(j):
                    sd[j] = pltpu.async_copy(bufs.at[j % 4], acc.at[dbuf.at[j]],
                                             ssem, add=True)

                for j in range(3):
                    fg(j)
                for j in range(ch):
                    gd[j].wait()
                    fs(j)
                    if j >= 1:
                        sd[j - 1].wait()
                    if j + 3 < ch:
                        fg(j + 3)
                sd[ch - 1].wait()
                return 0

            lax.fori_loop(0, n_chunks, _chunk, 0)
            plsc.subcore_barrier()
            for k in range(wo_n):
                pltpu.sync_copy(acc.at[pl.ds(s * rows_per_tile + k * wo_chunk, wo_chunk)],
                                tbuf.at[pl.ds(0, wo_chunk)])
                pltpu.sync_copy(tbuf.at[pl.ds(0, wo_chunk)],
                                outq.at[pl.ds(s * rows_per_tile + k * wo_chunk, wo_chunk)])

        for p in range(2):
            @pl.when(c == 0)
            def _qa():
                _run((z0, z2)[p], (o0, o2)[p])

            @pl.when(c == 1)
            def _qb():
                _run((z1, z3)[p], (o1, o3)[p])

    qt = jax.ShapeDtypeStruct((nodes, 32), jnp.float32)
    return pl.kernel(
        body,
        out_type=(qt, qt, qt, qt),
        mesh=plsc.VectorSubcoreMesh(
            core_axis_name="c", subcore_axis_name="s",
            num_cores=SC_CORES, num_subcores=SC_TILES),
        scratch_types=(
            pltpu.VMEM_SHARED((nodes, 32), jnp.float32),
            pltpu.VMEM((40, 128), jnp.int32),
            pltpu.VMEM((40, 128), jnp.int32),
            pltpu.VMEM((4, 128, 32), jnp.float32),
            pltpu.VMEM((wo_chunk, 32), jnp.float32),
            pltpu.SemaphoreType.DMA,
            pltpu.SemaphoreType.DMA,
        ),
        compiler_params=pltpu.CompilerParams(use_tc_tiling_on_sc=False),
    )


_drug_agg = _make_agg(NDP, 10, 112, 28)
_prot_agg = _make_agg(NCP, 8, 160, 16)


def _make_tc(n, bn):
    grid = (n // bn,)
    qspec = pl.BlockSpec((bn, 32), lambda i: (i, 0))
    fspec = pl.BlockSpec((bn, 128), lambda i: (i, 0))
    wspec = pl.BlockSpec((128, 128), lambda i: (0, 0))
    bspec = pl.BlockSpec((1, 128), lambda i: (0, 0))
    qt = jax.ShapeDtypeStruct((n, 32), jnp.float32)
    ft = jax.ShapeDtypeStruct((n, 128), jnp.float32)

    def pre_body(x_ref, W_ref, dis_ref, o0, o1, o2, o3):
        z = jnp.dot(x_ref[...], W_ref[...],
                    preferred_element_type=jnp.float32) * dis_ref[...]
        for q, o in enumerate((o0, o1, o2, o3)):
            o[...] = z[:, q * 32:(q + 1) * 32]

    pre = pl.pallas_call(pre_body, grid=grid,
                         in_specs=[fspec, wspec, fspec],
                         out_specs=(qspec,) * 4, out_shape=(qt,) * 4)

    def mid_body(s0, s1, s2, s3, z0, z1, z2, z3, dis_ref, b_ref, W_ref,
                 o0, o1, o2, o3):
        sagg = jnp.concatenate([s0[...], s1[...], s2[...], s3[...]], axis=1)
        z = jnp.concatenate([z0[...], z1[...], z2[...], z3[...]], axis=1)
        d = dis_ref[...]
        h = jnp.maximum(d * (sagg + z) + b_ref[...], 0.0)
        zn = jnp.dot(h, W_ref[...], preferred_element_type=jnp.float32) * d
        for q, o in enumerate((o0, o1, o2, o3)):
            o[...] = zn[:, q * 32:(q + 1) * 32]

    mid = pl.pallas_call(mid_body, grid=grid,
                         in_specs=[qspec] * 8 + [fspec, bspec, wspec],
                         out_specs=(qspec,) * 4, out_shape=(qt,) * 4)

    def fin_body(s0, s1, s2, s3, z0, z1, z2, z3, dis_ref, b_ref, h_out):
        sagg = jnp.concatenate([s0[...], s1[...], s2[...], s3[...]], axis=1)
        z = jnp.concatenate([z0[...], z1[...], z2[...], z3[...]], axis=1)
        h_out[...] = jnp.maximum(dis_ref[...] * (sagg + z) + b_ref[...], 0.0)

    fin = pl.pallas_call(fin_body, grid=grid,
                         in_specs=[qspec] * 8 + [fspec, bspec],
                         out_specs=fspec, out_shape=ft)
    return pre, mid, fin


_drug_pre, _drug_mid, _drug_fin = _make_tc(NDP, 3136)
_prot_pre, _prot_mid, _prot_fin = _make_tc(NCP, 2560)


def _pool_body(hd, hp, idsd, idsdc, idsp, idspc, dpool, ppool,
               acc, ibuf, ibufc, rowbuf, onesb, tbuf):
    c = lax.axis_index("c")
    s = lax.axis_index("s")

    def _ones(i, _):
        for g in range(8):
            onesb[i, pl.ds(g * 16, 16)] = jnp.full((16,), 1.0, jnp.float32)
        return 0

    def _zero(i, _):
        for g in range(8):
            tbuf[i, pl.ds(g * 16, 16)] = jnp.zeros((16,), jnp.float32)
        return 0

    lax.fori_loop(0, 128, _ones, 0)
    lax.fori_loop(0, 136, _zero, 0)
    pltpu.sync_copy(tbuf, acc.at[pl.ds(s * 136, 136)])
    plsc.subcore_barrier()

    @pl.when(c == 0)
    def _drug():
        nch = jnp.where(s == 0, 4, 3)
        base = jnp.where(s == 0, 0, 8 + s * 24)

        def ch(i, _):
            r0 = base + i * 8
            pltpu.sync_copy(idsd.at[pl.ds(r0, 8)], ibuf)
            pltpu.sync_copy(idsdc.at[pl.ds(r0, 8)], ibufc)
            for j in range(8):
                pltpu.sync_copy(hd.at[pl.ds((r0 + j) * 128, 128)], rowbuf)
                pltpu.sync_copy(rowbuf, acc.at[ibuf.at[j]], add=True)
                pltpu.sync_copy(onesb, acc.at[ibufc.at[j]], add=True)
            return 0

        lax.fori_loop(0, nch, ch, 0)

    @pl.when(c == 1)
    def _prot():
        nch = jnp.where(s < 8, 3, 2)
        base = jnp.where(s < 8, s * 24, 192 + (s - 8) * 16)

        def ch(i, _):
            r0 = base + i * 8
            pltpu.sync_copy(idsp.at[pl.ds(r0, 8)], ibuf)
            pltpu.sync_copy(idspc.at[pl.ds(r0, 8)], ibufc)
            for j in range(8):
                pltpu.sync_copy(hp.at[pl.ds((r0 + j) * 128, 128)], rowbuf)
                pltpu.sync_copy(rowbuf, acc.at[ibuf.at[j]], add=True)
                pltpu.sync_copy(onesb, acc.at[ibufc.at[j]], add=True)
            return 0

        lax.fori_loop(0, nch, ch, 0)

    plsc.subcore_barrier()

    @pl.when(c == 0)
    def _out_d():
        pltpu.sync_copy(acc.at[pl.ds(s * 40, 40)], tbuf.at[pl.ds(0, 40)])
        pltpu.sync_copy(tbuf.at[pl.ds(0, 40)], dpool.at[pl.ds(s * 40, 40)])

    @pl.when(c == 1)
    def _out_p():
        pltpu.sync_copy(acc.at[pl.ds(s * 136, 136)], tbuf)
        pltpu.sync_copy(tbuf, ppool.at[pl.ds(s * 136, 136)])


_pool_kernel = pl.kernel(
    _pool_body,
    out_type=(
        jax.ShapeDtypeStruct((640, 128), jnp.float32),
        jax.ShapeDtypeStruct((2176, 128), jnp.float32),
    ),
    mesh=plsc.VectorSubcoreMesh(
        core_axis_name="c", subcore_axis_name="s",
        num_cores=SC_CORES, num_subcores=SC_TILES),
    scratch_types=(
        pltpu.VMEM_SHARED((2176, 128), jnp.float32),
        pltpu.VMEM((8, 128), jnp.int32),
        pltpu.VMEM((8, 128), jnp.int32),
        pltpu.VMEM((128, 128), jnp.float32),
        pltpu.VMEM((128, 128), jnp.float32),
        pltpu.VMEM((136, 128), jnp.float32),
    ),
)


def _compute_degrees(dst_drug2d, dst_conf2d):
    return _deg_kernel(dst_drug2d, dst_conf2d)


def _tail_body(dpool_ref, ppool_ref, Wq_ref, bq_ref, Wk_ref, bk_ref, Wv_ref, bv_ref,
               Wh1_ref, bh1_ref, Wh2_ref, bh2_ref, logits_ref, attn_ref):
    drug = dpool_ref[0:256] / jnp.maximum(dpool_ref[320:576], 1.0)
    q = drug @ Wq_ref[...] + bq_ref[...]
    scores = []
    vals = []
    for c in range(NC):
        pc = (ppool_ref[c * 264:c * 264 + 256]
              / jnp.maximum(ppool_ref[1088 + c * 264:1088 + c * 264 + 256], 1.0))
        kc = pc @ Wk_ref[...] + bk_ref[...]
        vc = pc @ Wv_ref[...] + bv_ref[...]
        scores.append(jnp.sum(q * kc, axis=1) / (H ** 0.5))
        vals.append(vc)
    sc = jnp.stack(scores, axis=1)
    m = jnp.max(sc, axis=1, keepdims=True)
    e = jnp.exp(sc - m)
    attn = e / jnp.sum(e, axis=1, keepdims=True)
    attended = sum(vals[c] * attn[:, c:c + 1] for c in range(NC))
    h1 = jnp.maximum(drug @ Wh1_ref[:H] + attended @ Wh1_ref[H:] + bh1_ref[...], 0.0)
    logits_ref[...] = h1 @ Wh2_ref[...] + bh2_ref[...]
    attn_ref[...] = attn


def kernel(drug_x, drug_edge_index, drug_batch_ids, conf_x, conf_edge_index, conf_batch_ids,
           Wd0, bd0, Wd1, bd1, Wd2, bd2, Wp0, bp0, Wp1, bp1, Wp2, bp2,
           Wq, bq, Wk, bk, Wv, bv, Wh1, bh1, Wh2, bh2):
    src2d = jnp.pad(drug_edge_index[0], (0, ED_ROWS * 128 - ED),
                    constant_values=0).reshape(ED_ROWS, 128).astype(jnp.int32)
    dst2d = jnp.pad(drug_edge_index[1], (0, ED_ROWS * 128 - ED),
                    constant_values=NDP - 1).reshape(ED_ROWS, 128).astype(jnp.int32)
    psrc = conf_edge_index[:, 0, :] + (jnp.arange(NC, dtype=jnp.int32) * NPP)[:, None]
    psrc2d = jnp.pad(psrc, ((0, 0), (0, EP_ROWS * 128 - EP)),
                     constant_values=0).reshape(NC * EP_ROWS, 128).astype(jnp.int32)
    pdst_loc2d = jnp.pad(conf_edge_index[:, 1, :], ((0, 0), (0, EP_ROWS * 128 - EP)),
                         constant_values=NPP - 1).reshape(NC * EP_ROWS, 128).astype(jnp.int32)
    pdst = conf_edge_index[:, 1, :] + (jnp.arange(NC, dtype=jnp.int32) * NPP)[:, None]
    pdst2d = jnp.pad(pdst, ((0, 0), (0, EP_ROWS * 128 - EP)),
                     constant_values=NCP - 1).reshape(NC * EP_ROWS, 128).astype(jnp.int32)

    idsd2d = jnp.concatenate(
        [drug_batch_ids.astype(jnp.int32),
         jnp.full((NDP - ND,), 256, jnp.int32)]).reshape(NDP // 128, 128)
    idsdc2d = idsd2d + 320
    pb = conf_batch_ids.astype(jnp.int32)
    idsp2d = jnp.concatenate(
        [jnp.concatenate([pb + c * 264, jnp.full((NPP - NP_,), 1056, jnp.int32)])
         for c in range(NC)]).reshape(NCP // 128, 128)
    idspc2d = idsp2d + 1088

    deg_drug, deg_conf = _compute_degrees(dst2d, pdst_loc2d)
    dis_d2 = jnp.broadcast_to(lax.rsqrt(deg_drug + 1.0)[:, None], (NDP, 128))
    dis_p2 = jnp.broadcast_to(lax.rsqrt(deg_conf + 1.0)[:, None], (NCP, 128))

    x_pad = jnp.pad(drug_x, ((0, NDP - ND), (0, 128 - DD)))
    Wd0p = jnp.pad(Wd0, ((0, 128 - DD), (0, 0)))
    zq = _drug_pre(x_pad, Wd0p, dis_d2)
    for (b_prev, W) in ((bd0, Wd1), (bd1, Wd2)):
        sq = _drug_agg(*zq, src2d, dst2d)
        zq = _drug_mid(*sq, *zq, dis_d2, b_prev[None, :], W)
    sq = _drug_agg(*zq, src2d, dst2d)
    hd = _drug_fin(*sq, *zq, dis_d2, bd2[None, :])

    hp0 = jnp.pad(conf_x, ((0, 0), (0, NPP - NP_), (0, 0))).reshape(NCP, DP)
    zq = _prot_pre(hp0, Wp0, dis_p2)
    for (b_prev, W) in ((bp0, Wp1), (bp1, Wp2)):
        sq = _prot_agg(*zq, psrc2d, pdst2d)
        zq = _prot_mid(*sq, *zq, dis_p2, b_prev[None, :], W)
    sq = _prot_agg(*zq, psrc2d, pdst2d)
    hp = _prot_fin(*sq, *zq, dis_p2, bp2[None, :])

    dpool, ppool = _pool_kernel(hd, hp, idsd2d, idsdc2d, idsp2d, idspc2d)
    logits2, attn = pl.pallas_call(
        _tail_body,
        out_shape=(
            jax.ShapeDtypeStruct((B, 1), jnp.float32),
            jax.ShapeDtypeStruct((B, NC), jnp.float32),
        ),
    )(dpool, ppool, Wq, bq, Wk, bk, Wv, bv, Wh1, bh1, Wh2, bh2)
    return (logits2.squeeze(-1), attn)


# speedup vs baseline: 8.7750x; 1.0074x over previous
import functools

import jax
import jax.numpy as jnp
from jax import lax
from jax.experimental import pallas as pl
from jax.experimental.pallas import tpu as pltpu
from jax.experimental.pallas import tpu_sc as plsc

ND = 50000; ED = 800000; NP_ = 10000; EP = 160000; NC = 4; B = 256; DD = 78; DP = 128; H = 128

SC_CORES = 2
SC_TILES = 16

NDP = 50176
NPP = 10240
NCP = NC * NPP
ED_ROWS = 6400
EP_ROWS = 1280


def _deg_body(dst_drug, dst_conf, deg_drug, deg_conf, acc, idxbuf, ones_buf, zbuf, dsem):
    c = lax.axis_index("c")
    s = lax.axis_index("s")

    for k in range(8):
        ones_buf[pl.ds(k * 16, 16)] = jnp.full((16,), 1.0, jnp.float32)

    def _zero(i, _):
        zbuf[pl.ds(i * 16, 16)] = jnp.zeros((16,), jnp.float32)
        return 0

    lax.fori_loop(0, 3136 // 16, _zero, 0)

    pltpu.sync_copy(zbuf, acc.at[pl.ds(s * 3136, 3136)])
    plsc.subcore_barrier()

    @pl.when(c == 0)
    def _drug():
        def chunk(i, _):
            row0 = s * 400 + i * 16
            pltpu.sync_copy(dst_drug.at[pl.ds(row0, 16)], idxbuf)
            ds_ = [pltpu.async_copy(ones_buf, acc.at[idxbuf.at[j]], dsem, add=True)
                   for j in range(16)]
            for d in ds_:
                d.wait()
            return 0

        lax.fori_loop(0, 25, chunk, 0)

    @pl.when(c == 1)
    def _conf():
        conf = s // 4
        sub = s % 4
        off = conf * NPP

        def chunk(i, _):
            row0 = conf * EP_ROWS + sub * 320 + i * 16
            pltpu.sync_copy(dst_conf.at[pl.ds(row0, 16)], idxbuf)
            for j in range(16):
                for g in range(8):
                    v = idxbuf[j, pl.ds(g * 16, 16)]
                    idxbuf[j, pl.ds(g * 16, 16)] = v + off
            ds_ = [pltpu.async_copy(ones_buf, acc.at[idxbuf.at[j]], dsem, add=True)
                   for j in range(16)]
            for d in ds_:
                d.wait()
            return 0

        lax.fori_loop(0, 20, chunk, 0)

    plsc.subcore_barrier()

    @pl.when(c == 0)
    def _out_drug():
        pltpu.sync_copy(acc.at[pl.ds(s * 3136, 3136)], zbuf)
        pltpu.sync_copy(zbuf, deg_drug.at[pl.ds(s * 3136, 3136)])

    @pl.when(c == 1)
    def _out_conf():
        pltpu.sync_copy(acc.at[pl.ds(s * 2560, 2560)], zbuf.at[pl.ds(0, 2560)])
        pltpu.sync_copy(zbuf.at[pl.ds(0, 2560)], deg_conf.at[pl.ds(s * 2560, 2560)])


_deg_kernel = pl.kernel(
    _deg_body,
    out_type=(
        jax.ShapeDtypeStruct((NDP,), jnp.float32),
        jax.ShapeDtypeStruct((NCP,), jnp.float32),
    ),
    mesh=plsc.VectorSubcoreMesh(
        core_axis_name="c", subcore_axis_name="s",
        num_cores=SC_CORES, num_subcores=SC_TILES),
    scratch_types=(
        pltpu.VMEM_SHARED((NDP,), jnp.float32),
        pltpu.VMEM((16, 128), jnp.int32),
        pltpu.VMEM((128,), jnp.float32),
        pltpu.VMEM((3136,), jnp.float32),
        pltpu.SemaphoreType.DMA,
    ),
)


def _make_agg(nodes, n_chunks, wo_chunk, wo_n):
    rows_per_tile = wo_chunk * wo_n
    ch = 40

    def body(z0, z1, z2, z3, src2d, dst2d, o0, o1, o2, o3,
             acc, sbuf, dbuf, bufs, tbuf, gsem, ssem):
        c = lax.axis_index("c")
        s = lax.axis_index("s")

        def _zero_tbuf(i, _):
            for g in range(2):
                tbuf[i, pl.ds(g * 16, 16)] = jnp.zeros((16,), jnp.float32)
            return 0

        def _run(zq, outq):
            lax.fori_loop(0, wo_chunk, _zero_tbuf, 0)
            for k in range(wo_n):
                pltpu.sync_copy(tbuf.at[pl.ds(0, wo_chunk)],
                                acc.at[pl.ds(s * rows_per_tile + k * wo_chunk, wo_chunk)])
            plsc.subcore_barrier()

            def _chunk(i, _):
                row0 = s * (n_chunks * ch) + i * ch
                pltpu.sync_copy(src2d.at[pl.ds(row0, ch)], sbuf)
                pltpu.sync_copy(dst2d.at[pl.ds(row0, ch)], dbuf)
                gd = {}
                sd = {}

                def fg(j):
                    gd[j] = pltpu.async_copy(zq.at[sbuf.at[j]], bufs.at[j % 4], gsem)

                def fs(j):
                    sd[j] = pltpu.async_copy(bufs.at[j % 4], acc.at[dbuf.at[j]],
                                             ssem, add=True)

                for j in range(3):
                    fg(j)
                for j in range(ch):
                    gd[j].wait()
                    fs(j)
                    if j >= 1:
                        sd[j - 1].wait()
                    if j + 3 < ch:
                        fg(j + 3)
                sd[ch - 1].wait()
                return 0

            lax.fori_loop(0, n_chunks, _chunk, 0)
            plsc.subcore_barrier()
            for k in range(wo_n):
                pltpu.sync_copy(acc.at[pl.ds(s * rows_per_tile + k * wo_chunk, wo_chunk)],
                                tbuf.at[pl.ds(0, wo_chunk)])
                pltpu.sync_copy(tbuf.at[pl.ds(0, wo_chunk)],
                                outq.at[pl.ds(s * rows_per_tile + k * wo_chunk, wo_chunk)])

        for p in range(2):
            @pl.when(c == 0)
            def _qa():
                _run((z0, z2)[p], (o0, o2)[p])

            @pl.when(c == 1)
            def _qb():
                _run((z1, z3)[p], (o1, o3)[p])

    qt = jax.ShapeDtypeStruct((nodes, 32), jnp.float32)
    return pl.kernel(
        body,
        out_type=(qt, qt, qt, qt),
        mesh=plsc.VectorSubcoreMesh(
            core_axis_name="c", subcore_axis_name="s",
            num_cores=SC_CORES, num_subcores=SC_TILES),
        scratch_types=(
            pltpu.VMEM_SHARED((nodes, 32), jnp.float32),
            pltpu.VMEM((40, 128), jnp.int32),
            pltpu.VMEM((40, 128), jnp.int32),
            pltpu.VMEM((4, 128, 32), jnp.float32),
            pltpu.VMEM((wo_chunk, 32), jnp.float32),
            pltpu.SemaphoreType.DMA,
            pltpu.SemaphoreType.DMA,
        ),
        compiler_params=pltpu.CompilerParams(use_tc_tiling_on_sc=False),
    )


_drug_agg = _make_agg(NDP, 10, 112, 28)
_prot_agg = _make_agg(NCP, 8, 160, 16)


def _make_tc(n, bn):
    grid = (n // bn,)
    qspec = pl.BlockSpec((bn, 32), lambda i: (i, 0))
    fspec = pl.BlockSpec((bn, 128), lambda i: (i, 0))
    wspec = pl.BlockSpec((128, 128), lambda i: (0, 0))
    bspec = pl.BlockSpec((1, 128), lambda i: (0, 0))
    qt = jax.ShapeDtypeStruct((n, 32), jnp.float32)
    ft = jax.ShapeDtypeStruct((n, 128), jnp.float32)

    def pre_body(x_ref, W_ref, dis_ref, o0, o1, o2, o3):
        z = jnp.dot(x_ref[...], W_ref[...],
                    preferred_element_type=jnp.float32) * dis_ref[...]
        for q, o in enumerate((o0, o1, o2, o3)):
            o[...] = z[:, q * 32:(q + 1) * 32]

    pre = pl.pallas_call(pre_body, grid=grid,
                         in_specs=[fspec, wspec, fspec],
                         out_specs=(qspec,) * 4, out_shape=(qt,) * 4)

    def mid_body(s0, s1, s2, s3, z0, z1, z2, z3, dis_ref, b_ref, W_ref,
                 o0, o1, o2, o3):
        sagg = jnp.concatenate([s0[...], s1[...], s2[...], s3[...]], axis=1)
        z = jnp.concatenate([z0[...], z1[...], z2[...], z3[...]], axis=1)
        d = dis_ref[...]
        h = jnp.maximum(d * (sagg + z) + b_ref[...], 0.0)
        zn = jnp.dot(h, W_ref[...], preferred_element_type=jnp.float32) * d
        for q, o in enumerate((o0, o1, o2, o3)):
            o[...] = zn[:, q * 32:(q + 1) * 32]

    mid = pl.pallas_call(mid_body, grid=grid,
                         in_specs=[qspec] * 8 + [fspec, bspec, wspec],
                         out_specs=(qspec,) * 4, out_shape=(qt,) * 4)

    def fin_body(s0, s1, s2, s3, z0, z1, z2, z3, dis_ref, b_ref, h_out):
        sagg = jnp.concatenate([s0[...], s1[...], s2[...], s3[...]], axis=1)
        z = jnp.concatenate([z0[...], z1[...], z2[...], z3[...]], axis=1)
        h_out[...] = jnp.maximum(dis_ref[...] * (sagg + z) + b_ref[...], 0.0)

    fin = pl.pallas_call(fin_body, grid=grid,
                         in_specs=[qspec] * 8 + [fspec, bspec],
                         out_specs=fspec, out_shape=ft)
    return pre, mid, fin


_drug_pre, _drug_mid, _drug_fin = _make_tc(NDP, 3136)
_prot_pre, _prot_mid, _prot_fin = _make_tc(NCP, 2560)


def _pool_body(hd, hp, idsd, idsdc, idsp, idspc, dpool, ppool,
               acc, ibuf, ibufc, rowbuf, onesb, tbuf, gsem, ssem):
    c = lax.axis_index("c")
    s = lax.axis_index("s")

    def _ones(i, _):
        for g in range(8):
            onesb[i, pl.ds(g * 16, 16)] = jnp.full((16,), 1.0, jnp.float32)
        return 0

    def _zero(i, _):
        for g in range(8):
            tbuf[i, pl.ds(g * 16, 16)] = jnp.zeros((16,), jnp.float32)
        return 0

    lax.fori_loop(0, 128, _ones, 0)
    lax.fori_loop(0, 136, _zero, 0)
    pltpu.sync_copy(tbuf, acc.at[pl.ds(s * 136, 136)])
    plsc.subcore_barrier()

    def _graph_pool(hsrc, ids, idsc, nch, base):
        def ch(i, _):
            r0 = base + i * 8
            pltpu.sync_copy(ids.at[pl.ds(r0, 8)], ibuf)
            pltpu.sync_copy(idsc.at[pl.ds(r0, 8)], ibufc)
            gd = {}
            sd = {}

            def fg(j):
                gd[j] = pltpu.async_copy(hsrc.at[pl.ds((r0 + j) * 128, 128)],
                                         rowbuf.at[j % 4], gsem)

            for j in range(3):
                fg(j)
            for j in range(8):
                gd[j].wait()
                sd[j] = (
                    pltpu.async_copy(rowbuf.at[j % 4], acc.at[ibuf.at[j]], ssem,
                                     add=True),
                    pltpu.async_copy(onesb, acc.at[ibufc.at[j]], ssem, add=True),
                )
                if j >= 1:
                    sd[j - 1][0].wait()
                    sd[j - 1][1].wait()
                if j + 3 < 8:
                    fg(j + 3)
            sd[7][0].wait()
            sd[7][1].wait()
            return 0

        lax.fori_loop(0, nch, ch, 0)

    @pl.when(c == 0)
    def _drug():
        _graph_pool(hd, idsd, idsdc,
                    jnp.where(s == 0, 4, 3), jnp.where(s == 0, 0, 8 + s * 24))

    @pl.when(c == 1)
    def _prot():
        _graph_pool(hp, idsp, idspc,
                    jnp.where(s < 8, 3, 2), jnp.where(s < 8, s * 24, 192 + (s - 8) * 16))

    plsc.subcore_barrier()

    @pl.when(c == 0)
    def _out_d():
        pltpu.sync_copy(acc.at[pl.ds(s * 40, 40)], tbuf.at[pl.ds(0, 40)])
        pltpu.sync_copy(tbuf.at[pl.ds(0, 40)], dpool.at[pl.ds(s * 40, 40)])

    @pl.when(c == 1)
    def _out_p():
        pltpu.sync_copy(acc.at[pl.ds(s * 136, 136)], tbuf)
        pltpu.sync_copy(tbuf, ppool.at[pl.ds(s * 136, 136)])


_pool_kernel = pl.kernel(
    _pool_body,
    out_type=(
        jax.ShapeDtypeStruct((640, 128), jnp.float32),
        jax.ShapeDtypeStruct((2176, 128), jnp.float32),
    ),
    mesh=plsc.VectorSubcoreMesh(
        core_axis_name="c", subcore_axis_name="s",
        num_cores=SC_CORES, num_subcores=SC_TILES),
    scratch_types=(
        pltpu.VMEM_SHARED((2176, 128), jnp.float32),
        pltpu.VMEM((8, 128), jnp.int32),
        pltpu.VMEM((8, 128), jnp.int32),
        pltpu.VMEM((4, 128, 128), jnp.float32),
        pltpu.VMEM((128, 128), jnp.float32),
        pltpu.VMEM((136, 128), jnp.float32),
        pltpu.SemaphoreType.DMA,
        pltpu.SemaphoreType.DMA,
    ),
)


def _compute_degrees(dst_drug2d, dst_conf2d):
    return _deg_kernel(dst_drug2d, dst_conf2d)


def _tail_body(dpool_ref, ppool_ref, Wq_ref, bq_ref, Wk_ref, bk_ref, Wv_ref, bv_ref,
               Wh1_ref, bh1_ref, Wh2_ref, bh2_ref, logits_ref, attn_ref):
    drug = dpool_ref[0:256] / jnp.maximum(dpool_ref[320:576], 1.0)
    q = drug @ Wq_ref[...] + bq_ref[...]
    scores = []
    vals = []
    for c in range(NC):
        pc = (ppool_ref[c * 264:c * 264 + 256]
              / jnp.maximum(ppool_ref[1088 + c * 264:1088 + c * 264 + 256], 1.0))
        kc = pc @ Wk_ref[...] + bk_ref[...]
        vc = pc @ Wv_ref[...] + bv_ref[...]
        scores.append(jnp.sum(q * kc, axis=1) / (H ** 0.5))
        vals.append(vc)
    sc = jnp.stack(scores, axis=1)
    m = jnp.max(sc, axis=1, keepdims=True)
    e = jnp.exp(sc - m)
    attn = e / jnp.sum(e, axis=1, keepdims=True)
    attended = sum(vals[c] * attn[:, c:c + 1] for c in range(NC))
    h1 = jnp.maximum(drug @ Wh1_ref[:H] + attended @ Wh1_ref[H:] + bh1_ref[...], 0.0)
    logits_ref[...] = h1 @ Wh2_ref[...] + bh2_ref[...]
    attn_ref[...] = attn


def kernel(drug_x, drug_edge_index, drug_batch_ids, conf_x, conf_edge_index, conf_batch_ids,
           Wd0, bd0, Wd1, bd1, Wd2, bd2, Wp0, bp0, Wp1, bp1, Wp2, bp2,
           Wq, bq, Wk, bk, Wv, bv, Wh1, bh1, Wh2, bh2):
    src2d = jnp.pad(drug_edge_index[0], (0, ED_ROWS * 128 - ED),
                    constant_values=0).reshape(ED_ROWS, 128).astype(jnp.int32)
    dst2d = jnp.pad(drug_edge_index[1], (0, ED_ROWS * 128 - ED),
                    constant_values=NDP - 1).reshape(ED_ROWS, 128).astype(jnp.int32)
    psrc = conf_edge_index[:, 0, :] + (jnp.arange(NC, dtype=jnp.int32) * NPP)[:, None]
    psrc2d = jnp.pad(psrc, ((0, 0), (0, EP_ROWS * 128 - EP)),
                     constant_values=0).reshape(NC * EP_ROWS, 128).astype(jnp.int32)
    pdst_loc2d = jnp.pad(conf_edge_index[:, 1, :], ((0, 0), (0, EP_ROWS * 128 - EP)),
                         constant_values=NPP - 1).reshape(NC * EP_ROWS, 128).astype(jnp.int32)
    pdst = conf_edge_index[:, 1, :] + (jnp.arange(NC, dtype=jnp.int32) * NPP)[:, None]
    pdst2d = jnp.pad(pdst, ((0, 0), (0, EP_ROWS * 128 - EP)),
                     constant_values=NCP - 1).reshape(NC * EP_ROWS, 128).astype(jnp.int32)

    idsd2d = jnp.concatenate(
        [drug_batch_ids.astype(jnp.int32),
         jnp.full((NDP - ND,), 256, jnp.int32)]).reshape(NDP // 128, 128)
    idsdc2d = idsd2d + 320
    pb = conf_batch_ids.astype(jnp.int32)
    idsp2d = jnp.concatenate(
        [jnp.concatenate([pb + c * 264, jnp.full((NPP - NP_,), 1056, jnp.int32)])
         for c in range(NC)]).reshape(NCP // 128, 128)
    idspc2d = idsp2d + 1088

    deg_drug, deg_conf = _compute_degrees(dst2d, pdst_loc2d)
    dis_d2 = jnp.broadcast_to(lax.rsqrt(deg_drug + 1.0)[:, None], (NDP, 128))
    dis_p2 = jnp.broadcast_to(lax.rsqrt(deg_conf + 1.0)[:, None], (NCP, 128))

    x_pad = jnp.pad(drug_x, ((0, NDP - ND), (0, 128 - DD)))
    Wd0p = jnp.pad(Wd0, ((0, 128 - DD), (0, 0)))
    zq = _drug_pre(x_pad, Wd0p, dis_d2)
    for (b_prev, W) in ((bd0, Wd1), (bd1, Wd2)):
        sq = _drug_agg(*zq, src2d, dst2d)
        zq = _drug_mid(*sq, *zq, dis_d2, b_prev[None, :], W)
    sq = _drug_agg(*zq, src2d, dst2d)
    hd = _drug_fin(*sq, *zq, dis_d2, bd2[None, :])

    hp0 = jnp.pad(conf_x, ((0, 0), (0, NPP - NP_), (0, 0))).reshape(NCP, DP)
    zq = _prot_pre(hp0, Wp0, dis_p2)
    for (b_prev, W) in ((bp0, Wp1), (bp1, Wp2)):
        sq = _prot_agg(*zq, psrc2d, pdst2d)
        zq = _prot_mid(*sq, *zq, dis_p2, b_prev[None, :], W)
    sq = _prot_agg(*zq, psrc2d, pdst2d)
    hp = _prot_fin(*sq, *zq, dis_p2, bp2[None, :])

    dpool, ppool = _pool_kernel(hd, hp, idsd2d, idsdc2d, idsp2d, idspc2d)
    logits2, attn = pl.pallas_call(
        _tail_body,
        out_shape=(
            jax.ShapeDtypeStruct((B, 1), jnp.float32),
            jax.ShapeDtypeStruct((B, NC), jnp.float32),
        ),
    )(dpool, ppool, Wq, bq, Wk, bk, Wv, bv, Wh1, bh1, Wh2, bh2)
    return (logits2.squeeze(-1), attn)
